# baseline (speedup 1.0000x reference)
import functools

import jax
import jax.numpy as jnp
from jax import lax
from jax.experimental import pallas as pl
from jax.experimental.pallas import tpu as pltpu
from jax.experimental.pallas import tpu_sc as plsc

_B = 16384
_T = 200
_D = 64
_V = 100
_NW = 32
_BQ = _B // 4
_NT = _BQ // 128
_NG = _BQ // 16


def _sc_embedding_gather(ids_t, tab_x):
    mesh = plsc.VectorSubcoreMesh(core_axis_name="c", subcore_axis_name="s")

    @functools.partial(
        pl.kernel,
        out_type=jax.ShapeDtypeStruct((_T, 8, 128, 8, 128), jnp.float32),
        mesh=mesh,
        compiler_params=pltpu.CompilerParams(use_tc_tiling_on_sc=False,
                                             needs_layout_passes=False),
        scratch_types=[
            pltpu.VMEM((_V * _D,), jnp.float32),
            pltpu.VMEM((_BQ,), jnp.int32),
            pltpu.VMEM((_BQ,), jnp.int32),
            pltpu.VMEM((_NT, 8, 128), jnp.float32),
            pltpu.VMEM((_NT, 8, 128), jnp.float32),
            pltpu.SemaphoreType.DMA,
            pltpu.SemaphoreType.DMA,
            pltpu.SemaphoreType.DMA,
            pltpu.SemaphoreType.DMA,
        ],
    )
    def k(ids_hbm, tab_hbm, out_hbm, tab_v, iv0, iv1, ov0, ov1,
          si0, si1, so0, so1):
        wid = lax.axis_index("s") * 2 + lax.axis_index("c")
        d8 = wid // 4
        bq = wid % 4
        d8_off = d8 * (_V * 8)
        b_off = bq * _BQ
        iv = (iv0, iv1)
        ov = (ov0, ov1)
        sem_i = (si0, si1)
        sem_o = (so0, so1)

        pltpu.sync_copy(tab_hbm, tab_v)

        def ids_slice(t):
            return ids_hbm.at[t, pl.ds(b_off, _BQ)]

        def out_slice(t):
            return out_hbm.at[t, d8, pl.ds(bq * _NT, _NT)]

        def compute(b):
            iv_b, ov_b = iv[b], ov[b]

            def g_body(g, carry):
                ids16 = iv_b[pl.ds(g * 16, 16)]
                base = ids16 * 8 + d8_off
                btile = g // 8
                lane = (g % 8) * 16
                for d1 in range(8):
                    v = plsc.load_gather(tab_v, [base + d1])
                    ov_b[btile, d1, pl.ds(lane, 16)] = v
                return carry

            lax.fori_loop(0, _NG, g_body, 0)

        pltpu.async_copy(ids_slice(0), iv[0], sem_i[0])

        def body(t2, carry):
            for b in (0, 1):
                t = 2 * t2 + b
                b1 = 1 - b
                pltpu.make_async_copy(ids_slice(t), iv[b], sem_i[b]).wait()

                if b == 0:
                    pltpu.async_copy(ids_slice(t + 1), iv[b1], sem_i[b1])
                else:
                    @pl.when(t2 < _T // 2 - 1)
                    def _():
                        pltpu.async_copy(ids_slice(t + 1), iv[b1], sem_i[b1])

                @pl.when(t2 >= 1)
                def _():
                    pltpu.make_async_copy(ov[b], out_slice(t), sem_o[b]).wait()

                compute(b)
                pltpu.async_copy(ov[b], out_slice(t), sem_o[b])
            return carry

        lax.fori_loop(0, _T // 2, body, 0)

        pltpu.make_async_copy(ov[0], out_slice(0), sem_o[0]).wait()
        pltpu.make_async_copy(ov[1], out_slice(0), sem_o[1]).wait()

    return k(ids_t, tab_x)


def kernel(input_ids, embed_table):
    ids_t = input_ids.T.astype(jnp.int32)
    tab_x = (embed_table.reshape(_V, 8, 8).transpose(1, 0, 2)
             .reshape(_V * _D))
    out5 = _sc_embedding_gather(ids_t, tab_x)
    return out5.transpose(2, 4, 0, 1, 3).reshape(_B, _T, _D)


# speedup vs baseline: 6.9983x; 6.9983x over previous
import functools

import jax
import jax.numpy as jnp
from jax import lax
from jax.experimental import pallas as pl
from jax.experimental.pallas import tpu as pltpu
from jax.experimental.pallas import tpu_sc as plsc

_B = 16384
_T = 200
_D = 64
_V = 100
_BQ = _B // 4
_NT = _BQ // 128


def _sc_embedding_gather(ids_t, tab_x):
    mesh = plsc.VectorSubcoreMesh(core_axis_name="c", subcore_axis_name="s")

    @functools.partial(
        pl.kernel,
        out_type=jax.ShapeDtypeStruct((_T, 8, 128, 8, 128), jnp.float32),
        mesh=mesh,
        compiler_params=pltpu.CompilerParams(use_tc_tiling_on_sc=False,
                                             needs_layout_passes=False),
        scratch_types=[
            pltpu.VMEM((_V * _D,), jnp.float32),
            pltpu.VMEM((_BQ,), jnp.int32),
            pltpu.VMEM((_BQ,), jnp.int32),
            pltpu.VMEM((_NT, 8, 128), jnp.float32),
            pltpu.VMEM((_NT, 8, 128), jnp.float32),
            pltpu.SemaphoreType.DMA,
            pltpu.SemaphoreType.DMA,
            pltpu.SemaphoreType.DMA,
            pltpu.SemaphoreType.DMA,
        ],
    )
    def k(ids_hbm, tab_hbm, out_hbm, tab_v, iv0, iv1, ov0, ov1,
          si0, si1, so0, so1):
        wid = lax.axis_index("s") * 2 + lax.axis_index("c")
        d8 = wid // 4
        bq = wid % 4
        d8_off = d8 * _V
        b_off = bq * _BQ
        iv = (iv0, iv1)
        ov = (ov0, ov1)
        sem_i = (si0, si1)
        sem_o = (so0, so1)

        pltpu.sync_copy(tab_hbm, tab_v)

        def ids_slice(t):
            return ids_hbm.at[t, pl.ds(b_off, _BQ)]

        def out_slice(t):
            return out_hbm.at[t, d8, pl.ds(bq * _NT, _NT)]

        def compute(b):
            iv_b, ov_b = iv[b], ov[b]

            def t_body(btile, carry):
                ib = btile * 128
                bases = [iv_b[pl.ds(ib + gm * 16, 16)] + d8_off
                         for gm in range(8)]
                prev = None
                for d1 in range(8):
                    tab_d1 = tab_v.at[pl.ds(d1 * 8 * _V, 8 * _V)]
                    cur = []
                    for gm in range(8):
                        cur.append(plsc.load_gather(tab_d1, [bases[gm]]))
                        if prev is not None:
                            ov_b[btile, d1 - 1, pl.ds(gm * 16, 16)] = prev[gm]
                    prev = cur
                for gm in range(8):
                    ov_b[btile, 7, pl.ds(gm * 16, 16)] = prev[gm]
                return carry

            lax.fori_loop(0, _NT, t_body, 0)

        pltpu.async_copy(ids_slice(0), iv[0], sem_i[0])

        def body(t2, carry):
            for b in (0, 1):
                t = 2 * t2 + b
                b1 = 1 - b
                pltpu.make_async_copy(ids_slice(t), iv[b], sem_i[b]).wait()

                if b == 0:
                    pltpu.async_copy(ids_slice(t + 1), iv[b1], sem_i[b1])
                else:
                    @pl.when(t2 < _T // 2 - 1)
                    def _():
                        pltpu.async_copy(ids_slice(t + 1), iv[b1], sem_i[b1])

                @pl.when(t2 >= 1)
                def _():
                    pltpu.make_async_copy(ov[b], out_slice(t), sem_o[b]).wait()

                compute(b)
                pltpu.async_copy(ov[b], out_slice(t), sem_o[b])
            return carry

        lax.fori_loop(0, _T // 2, body, 0)

        pltpu.make_async_copy(ov[0], out_slice(0), sem_o[0]).wait()
        pltpu.make_async_copy(ov[1], out_slice(0), sem_o[1]).wait()

    return k(ids_t, tab_x)


def kernel(input_ids, embed_table):
    ids_t = input_ids.T.astype(jnp.int32)
    tab_x = (embed_table.reshape(_V, 8, 8).transpose(2, 1, 0)
             .reshape(_V * _D))
    out5 = _sc_embedding_gather(ids_t, tab_x)
    return out5.transpose(2, 4, 0, 1, 3).reshape(_B, _T, _D)
